# Initial kernel scaffold; baseline (speedup 1.0000x reference)
import functools
import math

import jax
import jax.numpy as jnp
from jax import lax
from jax.experimental import pallas as pl
from jax.experimental.pallas import tpu as pltpu
from jax.experimental.pallas import tpu_sc as plsc

D = 64
LANES = 16
CH = 512
IDX_BLK = 128
NC = 2
NS = 16
NW = NC * NS


def _sc_body(scale, n_rows, tok_hbm, x_hbm, seg_hbm, c_hbm, out_hbm,
             xidx, segb, cidx, tokbuf, cbuf, sem):
    wid = lax.axis_index("s") * NC + lax.axis_index("c")
    rows_per_w = n_rows // NW
    n_chunks = rows_per_w // CH
    iota = lax.iota(jnp.int32, LANES)

    @pl.loop(0, n_chunks)
    def _chunk(c):
        base = wid * rows_per_w + c * CH

        for j in range(CH // IDX_BLK):
            pltpu.sync_copy(x_hbm.at[pl.ds(base + j * IDX_BLK, IDX_BLK)],
                            xidx.at[j])
        pltpu.sync_copy(seg_hbm.at[pl.ds(base, CH)], segb)

        for i in range(CH // LANES):
            s16 = segb[pl.ds(i * LANES, LANES)]
            j, off = divmod(i * LANES, IDX_BLK)
            cidx[j, pl.ds(off, LANES)] = s16 * CH + (i * LANES + iota)

        copies = []
        for j in range(CH // IDX_BLK):
            sl = pl.ds(j * IDX_BLK, IDX_BLK)
            copies.append(
                pltpu.async_copy(tok_hbm.at[xidx.at[j]], tokbuf.at[sl], sem))
            copies.append(
                pltpu.async_copy(c_hbm.at[cidx.at[j]], cbuf.at[sl], sem))
        for cp in copies:
            cp.wait()

        @pl.loop(0, CH)
        def _row(r):
            for k in range(D // LANES):
                sl = pl.ds(k * LANES, LANES)
                tokbuf[r, sl] = tokbuf[r, sl] * scale + cbuf[r, sl]

        pltpu.sync_copy(tokbuf, out_hbm.at[pl.ds(base, CH)])


@functools.partial(jax.jit, static_argnames=("n_rows",))
def _sc_embed(token_table, x_flat, seg_flat, comb, n_rows):
    scale = float(math.sqrt(D))
    mesh = plsc.VectorSubcoreMesh(core_axis_name="c", subcore_axis_name="s")
    grid_kernel = pl.kernel(
        functools.partial(_sc_body, scale, n_rows),
        out_type=jax.ShapeDtypeStruct((n_rows, D), jnp.float32),
        mesh=mesh,
        scratch_types=[
            pltpu.VMEM((CH // IDX_BLK, IDX_BLK), jnp.int32),
            pltpu.VMEM((CH,), jnp.int32),
            pltpu.VMEM((CH // IDX_BLK, IDX_BLK), jnp.int32),
            pltpu.VMEM((CH, D), jnp.float32),
            pltpu.VMEM((CH, D), jnp.float32),
            pltpu.SemaphoreType.DMA,
        ],
    )
    return grid_kernel(token_table, x_flat, seg_flat, comb)


def kernel(x, segment_info, token_table, pos_embedding, segment_table):
    B, S = x.shape
    n_rows = B * S
    assert S == CH and n_rows % (NW * CH) == 0
    x_flat = x.reshape(n_rows).astype(jnp.int32)
    seg_flat = segment_info.reshape(n_rows).astype(jnp.int32)
    comb = (pos_embedding[0, :S, :][None, :, :]
            + segment_table[:, None, :]).reshape(-1, D)
    out = _sc_embed(token_table, x_flat, seg_flat, comb, n_rows)
    return out.reshape(B, S, D)


# speedup vs baseline: 5.1866x; 5.1866x over previous
import functools
import math

import jax
import jax.numpy as jnp
from jax import lax
from jax.experimental import pallas as pl
from jax.experimental.pallas import tpu as pltpu
from jax.experimental.pallas import tpu_sc as plsc

D = 64
LANES = 16
CH = 512
IDX_BLK = 128
NC = 2
NS = 16
NW = NC * NS


def _sc_body(scale, n_rows, tok_hbm, x_hbm, seg_hbm, c_hbm, out_hbm,
             xidx, segb, cidx, tokbuf, cbuf, sem):
    wid = lax.axis_index("s") * NC + lax.axis_index("c")
    rows_per_w = n_rows // NW
    n_chunks = rows_per_w // CH
    iota = lax.iota(jnp.int32, LANES)

    @pl.loop(0, n_chunks)
    def _chunk(c):
        base = wid * rows_per_w + c * CH

        for j in range(CH // IDX_BLK):
            pltpu.sync_copy(x_hbm.at[pl.ds(base + j * IDX_BLK, IDX_BLK)],
                            xidx.at[j])
        pltpu.sync_copy(seg_hbm.at[pl.ds(base, CH)], segb)

        for i in range(CH // LANES):
            s16 = segb[pl.ds(i * LANES, LANES)]
            j, off = divmod(i * LANES, IDX_BLK)
            cidx[j, pl.ds(off, LANES)] = s16 * CH + (i * LANES + iota)

        copies = []
        for j in range(CH // IDX_BLK):
            sl = pl.ds(j * IDX_BLK, IDX_BLK)
            copies.append(
                pltpu.async_copy(tok_hbm.at[xidx.at[j]], tokbuf.at[sl], sem))
            copies.append(
                pltpu.async_copy(c_hbm.at[cidx.at[j]], cbuf.at[sl], sem))
        for cp in copies:
            cp.wait()

        @pl.loop(0, CH)
        def _row(r):
            for k in range(D // LANES):
                sl = pl.ds(k * LANES, LANES)
                tokbuf[r, sl] = tokbuf[r, sl] * scale + cbuf[r, sl]

        pltpu.sync_copy(tokbuf, out_hbm.at[pl.ds(base, CH)])


@functools.partial(jax.jit, static_argnames=("n_rows",))
def _sc_embed(token_table, x_flat, seg_flat, comb, n_rows):
    scale = float(math.sqrt(D))
    mesh = plsc.VectorSubcoreMesh(core_axis_name="c", subcore_axis_name="s")
    grid_kernel = pl.kernel(
        functools.partial(_sc_body, scale, n_rows),
        out_type=jax.ShapeDtypeStruct((n_rows, D), jnp.float32),
        mesh=mesh,
        compiler_params=pltpu.CompilerParams(use_tc_tiling_on_sc=False),
        scratch_types=[
            pltpu.VMEM((CH // IDX_BLK, IDX_BLK), jnp.int32),
            pltpu.VMEM((CH,), jnp.int32),
            pltpu.VMEM((CH // IDX_BLK, IDX_BLK), jnp.int32),
            pltpu.VMEM((CH, D), jnp.float32),
            pltpu.VMEM((CH, D), jnp.float32),
            pltpu.SemaphoreType.DMA,
        ],
    )
    return grid_kernel(token_table, x_flat, seg_flat, comb)


def kernel(x, segment_info, token_table, pos_embedding, segment_table):
    B, S = x.shape
    n_rows = B * S
    assert S == CH and n_rows % (NW * CH) == 0
    x_flat = x.reshape(n_rows).astype(jnp.int32)
    seg_flat = segment_info.reshape(n_rows).astype(jnp.int32)
    comb = (pos_embedding[0, :S, :][None, :, :]
            + segment_table[:, None, :]).reshape(-1, D)
    out = _sc_embed(token_table, x_flat, seg_flat, comb, n_rows)
    return out.reshape(B, S, D)
